# baseline (speedup 1.0000x reference)
import jax
import jax.numpy as jnp
from jax.experimental import pallas as pl
from jax.experimental.pallas import tpu as pltpu

B = 1024
D = 64
LENS = (20, 50, 100, 200)
GRID = 10
LCS = tuple(L // GRID for L in LENS)


def _tc_body(x0, x1, x2, x3, o_ref):
    g = pl.program_id(0)

    @pl.when(g == 0)
    def _():
        o_ref[...] = jnp.zeros_like(o_ref)

    for i, x in enumerate((x0, x1, x2, x3)):
        o_ref[i, :, :] += jnp.sum(x[...], axis=0)


def kernel(inputs_0, inputs_1, inputs_2, inputs_3, sum_dim, concat_mode,
           keep_dims, cat_axis, is_cat):
    xs = (inputs_0, inputs_1, inputs_2, inputs_3)
    xt = [jnp.transpose(t, (1, 2, 0)) for t in xs]
    nb = 2
    bw = B // nb
    out = pl.pallas_call(
        _tc_body,
        grid=(nb, GRID),
        in_specs=[
            pl.BlockSpec((lc, D, bw), lambda h, g: (g, 0, h))
            for lc in LCS
        ],
        out_specs=pl.BlockSpec((4, D, bw), lambda h, g: (0, 0, h)),
        out_shape=jax.ShapeDtypeStruct((4, D, B), jnp.float32),
        compiler_params=pltpu.CompilerParams(
            dimension_semantics=("arbitrary", "arbitrary"),
        ),
    )(*xt)
    return out.transpose(2, 0, 1)
